# baseline (speedup 1.0000x reference)
import functools

import jax
import jax.numpy as jnp
from jax import lax
from jax.experimental import pallas as pl
from jax.experimental.pallas import tpu as pltpu
from jax.experimental.pallas import tpu_sc as plsc

NC = 2
NS = 16
NW = NC * NS
LANES = 16

DIM = 64
BB = 128
SUB = 8
TILE_WORDS = SUB * BB

NBUF = 4
DEPTH = 3
TBUF = 2


def _make_gather(batch: int, seq: int):
  assert batch == NW * BB and DIM == SUB * SUB
  n_ti = DIM // SUB

  mesh = plsc.VectorSubcoreMesh(core_axis_name="c", subcore_axis_name="s")

  @functools.partial(
      pl.kernel,
      out_type=jax.ShapeDtypeStruct((seq, n_ti, NW, TILE_WORDS), jnp.float32),
      mesh=mesh,
      scratch_types=[
          pltpu.VMEM((seq, BB), jnp.int32),
          pltpu.VMEM((NBUF, BB, DIM), jnp.float32),
          pltpu.VMEM((TBUF, DIM * BB), jnp.float32),
          pltpu.SemaphoreType.DMA((NBUF,)),
          pltpu.SemaphoreType.DMA((TBUF,)),
          pltpu.SemaphoreType.DMA,
      ],
      compiler_params=pltpu.CompilerParams(
          use_tc_tiling_on_sc=False, needs_layout_passes=False),
  )
  def gather_kernel(table_hbm, idx_hbm, out_hbm, idx_v, rows_v, tile_v,
                    gsem, osem, isem):
    wid = lax.axis_index("s") * NC + lax.axis_index("c")
    pltpu.sync_copy(idx_hbm.at[wid], idx_v)

    def gather_copy(l, slot):
      return pltpu.make_async_copy(
          table_hbm.at[idx_v.at[l]], rows_v.at[slot], gsem.at[slot])

    def out_copy(l, tslot, ti):
      return pltpu.make_async_copy(
          tile_v.at[tslot, pl.ds(ti * TILE_WORDS, TILE_WORDS)],
          out_hbm.at[l, ti, wid],
          osem.at[tslot])

    for l in range(DEPTH):
      gather_copy(l, l).start()

    base = lax.iota(jnp.int32, LANES) * BB

    def transpose_rows(slot, tslot):
      dst = tile_v.at[tslot]

      def body(bb, carry):
        src = rows_v.at[slot, bb]
        for k in range(DIM // LANES):
          v = src[pl.ds(k * LANES, LANES)]
          plsc.store_scatter(dst, [base + (bb + k * LANES * BB)], v)
        return carry

      lax.fori_loop(0, BB, body, 0)

    def step(l, carry):
      slot = lax.rem(l, NBUF)
      tslot = lax.rem(l, TBUF)
      gather_copy(l, slot).wait()

      @pl.when(l >= TBUF)
      def _():
        for ti in range(n_ti):
          out_copy(l - TBUF, tslot, ti).wait()

      transpose_rows(slot, tslot)
      for ti in range(n_ti):
        out_copy(l, tslot, ti).start()

      ln = l + DEPTH

      @pl.when(ln < seq)
      def _():
        gather_copy(ln, lax.rem(ln, NBUF)).start()

      return carry

    lax.fori_loop(0, seq, step, 0)

    for t in range(TBUF):
      l = seq - TBUF + t
      for ti in range(n_ti):
        out_copy(l, lax.rem(l, TBUF), ti).wait()

  return gather_kernel


def kernel(x, table):
  b, l = x.shape
  idx3 = x.reshape(NW, BB, l).transpose(0, 2, 1).astype(jnp.int32)
  table_flat = jax.lax.optimization_barrier(table.reshape(-1))
  table_lin = table_flat.reshape(table.shape)
  out = _make_gather(b, l)(table_lin, idx3)
  out5 = out.reshape(l, DIM // SUB, NW, SUB, BB)
  return out5.transpose(2, 4, 0, 1, 3).reshape(b, l, DIM)


# speedup vs baseline: 1.2170x; 1.2170x over previous
import functools

import jax
import jax.numpy as jnp
from jax import lax
from jax.experimental import pallas as pl
from jax.experimental.pallas import tpu as pltpu
from jax.experimental.pallas import tpu_sc as plsc

NC = 2
NS = 16
NW = NC * NS
LANES = 16

DIM = 64
BB = 128
SUB = 8
TILE_WORDS = SUB * BB

NBUF = 4
DEPTH = 3
TBUF = 2


def _make_gather(batch: int, seq: int):
  assert batch == NW * BB and DIM == SUB * SUB
  n_ti = DIM // SUB

  mesh = plsc.VectorSubcoreMesh(core_axis_name="c", subcore_axis_name="s")

  @functools.partial(
      pl.kernel,
      out_type=jax.ShapeDtypeStruct((seq, n_ti, NW, TILE_WORDS), jnp.float32),
      mesh=mesh,
      scratch_types=[
          pltpu.VMEM((seq, BB), jnp.int32),
          pltpu.VMEM((NBUF, BB, DIM), jnp.float32),
          pltpu.VMEM((TBUF, DIM * BB), jnp.float32),
          pltpu.SemaphoreType.DMA((NBUF,)),
          pltpu.SemaphoreType.DMA((TBUF,)),
          pltpu.SemaphoreType.DMA,
      ],
      compiler_params=pltpu.CompilerParams(
          use_tc_tiling_on_sc=False, needs_layout_passes=False),
  )
  def gather_kernel(table_hbm, idx_hbm, out_hbm, idx_v, rows_v, tile_v,
                    gsem, osem, isem):
    wid = lax.axis_index("s") * NC + lax.axis_index("c")
    pltpu.sync_copy(idx_hbm.at[wid], idx_v)

    def gather_copy(l, slot):
      return pltpu.make_async_copy(
          table_hbm.at[idx_v.at[l]], rows_v.at[slot], gsem.at[slot])

    def out_copy(l, tslot, ti):
      return pltpu.make_async_copy(
          tile_v.at[tslot, pl.ds(ti * TILE_WORDS, TILE_WORDS)],
          out_hbm.at[l, ti, wid],
          osem.at[tslot])

    for l in range(DEPTH):
      gather_copy(l, l).start()

    iota = lax.iota(jnp.int32, LANES)
    bases = [iota * BB + k * LANES * BB for k in range(DIM // LANES)]

    def transpose_rows(slot, tslot):
      dst = tile_v.at[tslot]

      @plsc.parallel_loop(0, BB, 1, unroll=8)
      def _(bb):
        for k in range(DIM // LANES):
          v = rows_v[slot, bb, pl.ds(k * LANES, LANES)]
          plsc.store_scatter(dst, [bases[k] + bb], v)

    def step(l, carry):
      slot = lax.rem(l, NBUF)
      tslot = lax.rem(l, TBUF)
      gather_copy(l, slot).wait()

      @pl.when(l >= TBUF)
      def _():
        for ti in range(n_ti):
          out_copy(l - TBUF, tslot, ti).wait()

      transpose_rows(slot, tslot)
      for ti in range(n_ti):
        out_copy(l, tslot, ti).start()

      ln = l + DEPTH

      @pl.when(ln < seq)
      def _():
        gather_copy(ln, lax.rem(ln, NBUF)).start()

      return carry

    lax.fori_loop(0, seq, step, 0)

    for t in range(TBUF):
      l = seq - TBUF + t
      for ti in range(n_ti):
        out_copy(l, lax.rem(l, TBUF), ti).wait()

  return gather_kernel


def kernel(x, table):
  b, l = x.shape
  idx3 = x.reshape(NW, BB, l).transpose(0, 2, 1).astype(jnp.int32)
  table_flat = jax.lax.optimization_barrier(table.reshape(-1))
  table_lin = table_flat.reshape(table.shape)
  out = _make_gather(b, l)(table_lin, idx3)
  out5 = out.reshape(l, DIM // SUB, NW, SUB, BB)
  return out5.transpose(2, 4, 0, 1, 3).reshape(b, l, DIM)


# speedup vs baseline: 1.9869x; 1.6325x over previous
import functools

import jax
import jax.numpy as jnp
from jax import lax
from jax.experimental import pallas as pl
from jax.experimental.pallas import tpu as pltpu
from jax.experimental.pallas import tpu_sc as plsc

NC = 2
NS = 16
NW = NC * NS
LANES = 16

DIM = 64
BB = 128
SUB = 8
TILE_WORDS = SUB * BB
CW = BB * DIM

NBUF = 4
DEPTH = 3
TBUF = 2


def _make_gather(batch: int, seq: int):
  assert batch == NW * BB and DIM == SUB * SUB
  n_ti = DIM // SUB

  mesh = plsc.VectorSubcoreMesh(core_axis_name="c", subcore_axis_name="s")

  @functools.partial(
      pl.kernel,
      out_type=jax.ShapeDtypeStruct((seq, n_ti, NW, TILE_WORDS), jnp.float32),
      mesh=mesh,
      scratch_types=[
          pltpu.VMEM((seq, BB), jnp.int32),
          pltpu.VMEM((NBUF, BB, DIM), jnp.float32),
          pltpu.VMEM((TBUF * DIM * BB,), jnp.float32),
          pltpu.SemaphoreType.DMA((NBUF,)),
          pltpu.SemaphoreType.DMA((TBUF,)),
      ],
      compiler_params=pltpu.CompilerParams(
          use_tc_tiling_on_sc=False, needs_layout_passes=False,
          disable_bounds_checks=True),
  )
  def gather_kernel(table_hbm, idx_hbm, out_hbm, idx_v, rows_v, tile_v,
                    gsem, osem):
    wid = lax.axis_index("s") * NC + lax.axis_index("c")
    pltpu.sync_copy(idx_hbm.at[wid], idx_v)

    def gather_copy(l, slot):
      return pltpu.make_async_copy(
          table_hbm.at[idx_v.at[l]], rows_v.at[slot], gsem.at[slot])

    def out_copy(l, tslot, ti):
      return pltpu.make_async_copy(
          tile_v.at[pl.ds(tslot * DIM * BB + ti * TILE_WORDS, TILE_WORDS)],
          out_hbm.at[l, ti, wid],
          osem.at[tslot])

    for l in range(DEPTH):
      gather_copy(l, l).start()

    iota = lax.iota(jnp.int32, LANES)
    perms = [lax.rem(iota + t, LANES) for t in range(LANES)]
    dst_vecs = [iota * BB + perms[t] for t in range(LANES)]
    col_vecs = [iota + dblk * LANES for dblk in range(DIM // LANES)]

    def transpose_rows(slot, tslot):
      dst_base0 = tslot * DIM * BB
      src2 = rows_v.at[slot]

      for dblk in range(DIM // LANES):
        @plsc.parallel_loop(0, BB // LANES, 1, unroll=2)
        def _(g):
          bb0 = g * LANES
          d_base = dst_base0 + dblk * (LANES * BB) + bb0
          for t in range(LANES):
            v = plsc.load_gather(src2, [perms[t] + bb0, col_vecs[dblk]])
            plsc.store_scatter(tile_v, [dst_vecs[t] + d_base], v)

    def step(l, carry):
      slot = lax.rem(l, NBUF)
      tslot = lax.rem(l, TBUF)
      gather_copy(l, slot).wait()

      @pl.when(l >= TBUF)
      def _():
        for ti in range(n_ti):
          out_copy(l - TBUF, tslot, ti).wait()

      transpose_rows(slot, tslot)
      for ti in range(n_ti):
        out_copy(l, tslot, ti).start()

      ln = l + DEPTH

      @pl.when(ln < seq)
      def _():
        gather_copy(ln, lax.rem(ln, NBUF)).start()

      return carry

    lax.fori_loop(0, seq, step, 0)

    for t in range(TBUF):
      l = seq - TBUF + t
      for ti in range(n_ti):
        out_copy(l, lax.rem(l, TBUF), ti).wait()

  return gather_kernel


def kernel(x, table):
  b, l = x.shape
  idx3 = x.reshape(NW, BB, l).transpose(0, 2, 1).astype(jnp.int32)
  table_flat = jax.lax.optimization_barrier(table.reshape(-1))
  table_lin = table_flat.reshape(table.shape)
  out = _make_gather(b, l)(table_lin, idx3)
  out5 = out.reshape(l, DIM // SUB, NW, SUB, BB)
  return out5.transpose(2, 4, 0, 1, 3).reshape(b, l, DIM)


# speedup vs baseline: 1.9938x; 1.0035x over previous
import functools

import jax
import jax.numpy as jnp
from jax import lax
from jax.experimental import pallas as pl
from jax.experimental.pallas import tpu as pltpu
from jax.experimental.pallas import tpu_sc as plsc

NC = 2
NS = 16
NW = NC * NS
LANES = 16

DIM = 64
BB = 128
SUB = 8
TILE_WORDS = SUB * BB
CW = BB * DIM

NBUF = 6
DEPTH = 5
TBUF = 2


def _make_gather(batch: int, seq: int):
  assert batch == NW * BB and DIM == SUB * SUB
  n_ti = DIM // SUB

  mesh = plsc.VectorSubcoreMesh(core_axis_name="c", subcore_axis_name="s")

  @functools.partial(
      pl.kernel,
      out_type=jax.ShapeDtypeStruct((seq, n_ti, NW, TILE_WORDS), jnp.float32),
      mesh=mesh,
      scratch_types=[
          pltpu.VMEM((seq, BB), jnp.int32),
          pltpu.VMEM((NBUF, BB, DIM), jnp.float32),
          pltpu.VMEM((TBUF * DIM * BB,), jnp.float32),
          pltpu.SemaphoreType.DMA((NBUF,)),
          pltpu.SemaphoreType.DMA((TBUF,)),
      ],
      compiler_params=pltpu.CompilerParams(
          use_tc_tiling_on_sc=False, needs_layout_passes=False,
          disable_bounds_checks=True),
  )
  def gather_kernel(table_hbm, idx_hbm, out_hbm, idx_v, rows_v, tile_v,
                    gsem, osem):
    wid = lax.axis_index("s") * NC + lax.axis_index("c")
    pltpu.sync_copy(idx_hbm.at[wid], idx_v)

    def gather_copy(l, slot):
      return pltpu.make_async_copy(
          table_hbm.at[idx_v.at[l]], rows_v.at[slot], gsem.at[slot])

    def out_copy(l, tslot, ti):
      return pltpu.make_async_copy(
          tile_v.at[pl.ds(tslot * DIM * BB + ti * TILE_WORDS, TILE_WORDS)],
          out_hbm.at[l, ti, wid],
          osem.at[tslot])

    for l in range(DEPTH):
      gather_copy(l, l).start()

    iota = lax.iota(jnp.int32, LANES)
    perms = [lax.rem(iota + t, LANES) for t in range(LANES)]
    dst_vecs = [iota * BB + perms[t] for t in range(LANES)]
    col_vecs = [iota + dblk * LANES for dblk in range(DIM // LANES)]

    def transpose_rows(slot, tslot):
      dst_base0 = tslot * DIM * BB
      src2 = rows_v.at[slot]

      for dblk in range(DIM // LANES):
        @plsc.parallel_loop(0, BB // LANES, 1, unroll=2)
        def _(g):
          bb0 = g * LANES
          d_base = dst_base0 + dblk * (LANES * BB) + bb0
          for t in range(LANES):
            v = plsc.load_gather(src2, [perms[t] + bb0, col_vecs[dblk]])
            plsc.store_scatter(tile_v, [dst_vecs[t] + d_base], v)

    def step(l, carry):
      slot = lax.rem(l, NBUF)
      tslot = lax.rem(l, TBUF)
      gather_copy(l, slot).wait()

      @pl.when(l >= TBUF)
      def _():
        for ti in range(n_ti):
          out_copy(l - TBUF, tslot, ti).wait()

      transpose_rows(slot, tslot)
      for ti in range(n_ti):
        out_copy(l, tslot, ti).start()

      ln = l + DEPTH

      @pl.when(ln < seq)
      def _():
        gather_copy(ln, lax.rem(ln, NBUF)).start()

      return carry

    lax.fori_loop(0, seq, step, 0)

    for t in range(TBUF):
      l = seq - TBUF + t
      for ti in range(n_ti):
        out_copy(l, lax.rem(l, TBUF), ti).wait()

  return gather_kernel


def kernel(x, table):
  b, l = x.shape
  idx3 = x.reshape(NW, BB, l).transpose(0, 2, 1).astype(jnp.int32)
  table_flat = jax.lax.optimization_barrier(table.reshape(-1))
  table_lin = table_flat.reshape(table.shape)
  out = _make_gather(b, l)(table_lin, idx3)
  out5 = out.reshape(l, DIM // SUB, NW, SUB, BB)
  return out5.transpose(2, 4, 0, 1, 3).reshape(b, l, DIM)


# speedup vs baseline: 2.1321x; 1.0694x over previous
import functools

import jax
import jax.numpy as jnp
from jax import lax
from jax.experimental import pallas as pl
from jax.experimental.pallas import tpu as pltpu
from jax.experimental.pallas import tpu_sc as plsc

NC = 2
NS = 16
NW = NC * NS
LANES = 16

DIM = 64
BB = 128
SUB = 8
TILE_WORDS = SUB * BB
CW = BB * DIM

NBUF = 6
DEPTH = 5
TBUF = 3


def _make_gather(batch: int, seq: int):
  assert batch == NW * BB and DIM == SUB * SUB
  n_ti = DIM // SUB

  mesh = plsc.VectorSubcoreMesh(core_axis_name="c", subcore_axis_name="s")

  @functools.partial(
      pl.kernel,
      out_type=jax.ShapeDtypeStruct((seq, n_ti, NW, TILE_WORDS), jnp.float32),
      mesh=mesh,
      scratch_types=[
          pltpu.VMEM((seq, BB), jnp.int32),
          pltpu.VMEM((NBUF, BB, DIM), jnp.float32),
          pltpu.VMEM((TBUF * DIM * BB,), jnp.float32),
          pltpu.SemaphoreType.DMA((NBUF,)),
          pltpu.SemaphoreType.DMA((TBUF,)),
      ],
      compiler_params=pltpu.CompilerParams(
          use_tc_tiling_on_sc=False, needs_layout_passes=False,
          disable_bounds_checks=True),
  )
  def gather_kernel(table_hbm, idx_hbm, out_hbm, idx_v, rows_v, tile_v,
                    gsem, osem):
    wid = lax.axis_index("s") * NC + lax.axis_index("c")
    pltpu.sync_copy(idx_hbm.at[wid], idx_v)

    def gather_copy(l, slot):
      return pltpu.make_async_copy(
          table_hbm.at[idx_v.at[l]], rows_v.at[slot], gsem.at[slot])

    def out_copy(l, tslot, ti):
      return pltpu.make_async_copy(
          tile_v.at[pl.ds(tslot * DIM * BB + ti * TILE_WORDS, TILE_WORDS)],
          out_hbm.at[l, ti, wid],
          osem.at[tslot])

    for l in range(DEPTH):
      gather_copy(l, l).start()

    iota = lax.iota(jnp.int32, LANES)
    perms = [lax.rem(iota + t, LANES) for t in range(LANES)]
    dst_vecs = [iota * BB + perms[t] for t in range(LANES)]
    col_vecs = [iota + dblk * LANES for dblk in range(DIM // LANES)]

    def transpose_rows(slot, tslot):
      dst_base0 = tslot * DIM * BB
      src2 = rows_v.at[slot]

      for dblk in range(DIM // LANES):
        @plsc.parallel_loop(0, BB // LANES, 1, unroll=4)
        def _(g):
          bb0 = g * LANES
          d_base = dst_base0 + dblk * (LANES * BB) + bb0
          for t in range(LANES):
            v = plsc.load_gather(src2, [perms[t] + bb0, col_vecs[dblk]])
            plsc.store_scatter(tile_v, [dst_vecs[t] + d_base], v)

    def step(l, carry):
      slot = lax.rem(l, NBUF)
      tslot = lax.rem(l, TBUF)
      gather_copy(l, slot).wait()

      @pl.when(l >= TBUF)
      def _():
        for ti in range(n_ti):
          out_copy(l - TBUF, tslot, ti).wait()

      transpose_rows(slot, tslot)
      for ti in range(n_ti):
        out_copy(l, tslot, ti).start()

      ln = l + DEPTH

      @pl.when(ln < seq)
      def _():
        gather_copy(ln, lax.rem(ln, NBUF)).start()

      return carry

    lax.fori_loop(0, seq, step, 0)

    for t in range(TBUF):
      l = seq - TBUF + t
      for ti in range(n_ti):
        out_copy(l, lax.rem(l, TBUF), ti).wait()

  return gather_kernel


def kernel(x, table):
  b, l = x.shape
  idx3 = x.reshape(NW, BB, l).transpose(0, 2, 1).astype(jnp.int32)
  table_flat = jax.lax.optimization_barrier(table.reshape(-1))
  table_lin = table_flat.reshape(table.shape)
  out = _make_gather(b, l)(table_lin, idx3)
  out5 = out.reshape(l, DIM // SUB, NW, SUB, BB)
  return out5.transpose(2, 4, 0, 1, 3).reshape(b, l, DIM)
